# baseline (speedup 1.0000x reference)
import jax
import jax.numpy as jnp
from jax import lax
from jax.experimental import pallas as pl
from jax.experimental.pallas import tpu as pltpu
from jax.experimental.pallas import tpu_sc as plsc

EPS = 0.001
PROX = 0.15

E = 2048
TR = 256
TCOL = 256
SUB = 32
PT = 256
NF = 16

_NC = 2
_NS = 16
_NW = _NC * _NS
_EPW = E // _NW


def _sqrt16(x):
    g = plsc.bitcast(
        lax.shift_right_logical(plsc.bitcast(x, jnp.int32), 1) + 0x1FBD1DF6,
        jnp.float32)
    g = 0.5 * (g + x / g)
    g = 0.5 * (g + x / g)
    g = 0.5 * (g + x / g)
    return g


def _sc_prep(pos_hbm, edge_hbm, rows_hbm, pos_v, src_v, dst_v, rowbuf):
    wid = lax.axis_index("s") * _NC + lax.axis_index("c")
    base = wid * _EPW
    pltpu.sync_copy(pos_hbm, pos_v)
    pltpu.sync_copy(edge_hbm.at[pl.ds(base, _EPW)], src_v)
    pltpu.sync_copy(edge_hbm.at[pl.ds(E + base, _EPW)], dst_v)
    for c in range(_EPW // 16):
        sv = src_v[pl.ds(c * 16, 16)]
        dv = dst_v[pl.ds(c * 16, 16)]
        sx = plsc.load_gather(pos_v, [sv * 2])
        sy = plsc.load_gather(pos_v, [sv * 2 + 1])
        ex = plsc.load_gather(pos_v, [dv * 2])
        ey = plsc.load_gather(pos_v, [dv * 2 + 1])
        dx = ex - sx
        dy = ey - sy
        len2 = dx * dx + dy * dy
        a = jnp.maximum(len2, 1e-12)
        ra = 1.0 / a
        mx = (sx + ex) * 0.5
        my = (sy + ey) * 0.5
        h = 0.5 * _sqrt16(len2)
        sf = sv.astype(jnp.float32)
        df = dv.astype(jnp.float32)
        z = jnp.zeros((16,), jnp.float32)
        fields = [sx, sy, dx, dy, a, ra, mx, my, h, sf, df, z, z, z, z, z]
        eidx = (c * 16 + lax.broadcasted_iota(jnp.int32, (16,), 0)) * NF
        for fi, val in enumerate(fields):
            plsc.store_scatter(rowbuf, [eidx + fi], val)
    pltpu.sync_copy(rowbuf, rows_hbm.at[pl.ds(base * NF, _EPW * NF)])


def _body(rows_ref, out_ref, cols_s, accs_ref, accc_ref):
    i = pl.program_id(0)
    j = pl.program_id(1)

    @pl.when((i == 0) & (j == 0))
    def _init():
        for t in range(E // PT):
            cols_s[:, t * PT:(t + 1) * PT] = rows_ref[t * PT:(t + 1) * PT, :].T
        accs_ref[...] = jnp.zeros((SUB, TCOL), jnp.float32)
        accc_ref[...] = jnp.zeros((SUB, TCOL), jnp.float32)

    @pl.when(j >= i)
    def _compute():
        joff = pl.multiple_of(j * TCOL, TCOL)
        cb = cols_s[:, pl.ds(joff, TCOL)]
        sxj = cb[0:1, :]
        syj = cb[1:2, :]
        dxj = cb[2:3, :]
        dyj = cb[3:4, :]
        ej = cb[4:5, :]
        rej = cb[5:6, :]
        mxj = cb[6:7, :]
        myj = cb[7:8, :]
        hj = cb[8:9, :]
        sfj = cb[9:10, :]
        dfj = cb[10:11, :]
        col_ids = j * TCOL + lax.broadcasted_iota(jnp.int32, (1, TCOL), 1)

        tp = jnp.zeros((SUB, TCOL), jnp.float32)
        tc = jnp.zeros((SUB, TCOL), jnp.float32)
        for k in range(TR // SUB):
            rb = rows_ref[pl.ds(i * TR + k * SUB, SUB), :]
            sxi = rb[:, 0:1]
            syi = rb[:, 1:2]
            dxi = rb[:, 2:3]
            dyi = rb[:, 3:4]
            ai = rb[:, 4:5]
            rai = rb[:, 5:6]
            mxi = rb[:, 6:7]
            myi = rb[:, 7:8]
            hi = rb[:, 8:9]
            sfi = rb[:, 9:10]
            dfi = rb[:, 10:11]

            adj = ((sfi == sfj) | (sfi == dfj) | (dfi == sfj) | (dfi == dfj))
            row_ids = (i * TR + k * SUB
                       + lax.broadcasted_iota(jnp.int32, (SUB, 1), 0))
            tri = col_ids > row_ids
            mdx = mxi - mxj
            mdy = myi - myj
            md2 = mdx * mdx + mdy * mdy
            prox = hi + hj + PROX
            mask = (~adj) & tri & (md2 < prox * prox)
            maskf = mask.astype(jnp.float32)

            b = dxi * dxj + dyi * dyj
            rx = sxi - sxj
            ry = syi - syj
            c = dxi * rx + dyi * ry
            f = dxj * rx + dyj * ry
            rdenom = 1.0 / jnp.maximum(ai * ej - b * b, 1e-12)
            s = jnp.clip((b * f - c * ej) * rdenom, 0.0, 1.0)
            t = jnp.clip((b * s + f) * rej, 0.0, 1.0)
            s = jnp.clip((b * t - c) * rai, 0.0, 1.0)
            ddx = rx + s * dxi - t * dxj
            ddy = ry + s * dyi - t * dyj
            sq = ddx * ddx + ddy * ddy
            dist = jnp.sqrt(jnp.maximum(sq, 1e-24))
            per = jnp.maximum(EPS - dist, 0.0) * maskf

            tp = tp + per
            tc = tc + maskf
        accs_ref[...] += tp
        accc_ref[...] += tc

    @pl.when((i == pl.num_programs(0) - 1) & (j == pl.num_programs(1) - 1))
    def _final():
        total = jnp.sum(accs_ref[...])
        cnt = jnp.sum(accc_ref[...])
        loss = jnp.where(cnt > 0.0, total / jnp.maximum(cnt, 1.0), 0.0)
        out_ref[...] = loss.reshape(1, 1)


def kernel(node_positions, adjacency, edge_index, weight):
    del adjacency, weight
    pos_flat = node_positions.reshape(-1)
    edge_flat = edge_index.reshape(-1)

    sc_prep = pl.kernel(
        _sc_prep,
        out_type=jax.ShapeDtypeStruct((E * NF,), jnp.float32),
        mesh=plsc.VectorSubcoreMesh(
            core_axis_name="c", subcore_axis_name="s",
            num_cores=_NC, num_subcores=_NS),
        scratch_types=[
            pltpu.VMEM((2 * E,), jnp.float32),
            pltpu.VMEM((_EPW,), jnp.int32),
            pltpu.VMEM((_EPW,), jnp.int32),
            pltpu.VMEM((_EPW * NF,), jnp.float32),
        ],
        compiler_params=pltpu.CompilerParams(
            needs_layout_passes=False, use_tc_tiling_on_sc=False),
    )
    rows = sc_prep(pos_flat, edge_flat).reshape(E, NF)

    loss = pl.pallas_call(
        _body,
        grid=(E // TR, E // TCOL),
        in_specs=[pl.BlockSpec((E, NF), lambda i, j: (0, 0))],
        out_specs=pl.BlockSpec((1, 1), lambda i, j: (0, 0)),
        out_shape=jax.ShapeDtypeStruct((1, 1), jnp.float32),
        scratch_shapes=[
            pltpu.VMEM((NF, E), jnp.float32),
            pltpu.VMEM((SUB, TCOL), jnp.float32),
            pltpu.VMEM((SUB, TCOL), jnp.float32),
        ],
    )(rows)
    return loss.reshape(())


# speedup vs baseline: 1.0009x; 1.0009x over previous
import jax
import jax.numpy as jnp
from jax import lax
from jax.experimental import pallas as pl
from jax.experimental.pallas import tpu as pltpu
from jax.experimental.pallas import tpu_sc as plsc

EPS = 0.001
PROX = 0.15

E = 2048
TR = 256
TCOL = 256
SUB = 32
PT = 256
NF = 16

_NC = 2
_NS = 16
_NW = _NC * _NS
_EPW = E // _NW


def _sqrt16(x):
    g = plsc.bitcast(
        lax.shift_right_logical(plsc.bitcast(x, jnp.int32), 1) + 0x1FBD1DF6,
        jnp.float32)
    g = 0.5 * (g + x / g)
    g = 0.5 * (g + x / g)
    g = 0.5 * (g + x / g)
    return g


def _sc_prep(pos_hbm, edge_hbm, rows_hbm, pos_v, src_v, dst_v, rowbuf):
    wid = lax.axis_index("s") * _NC + lax.axis_index("c")
    base = wid * _EPW
    pltpu.sync_copy(pos_hbm, pos_v)
    pltpu.sync_copy(edge_hbm.at[pl.ds(base, _EPW)], src_v)
    pltpu.sync_copy(edge_hbm.at[pl.ds(E + base, _EPW)], dst_v)
    for c in range(_EPW // 16):
        sv = src_v[pl.ds(c * 16, 16)]
        dv = dst_v[pl.ds(c * 16, 16)]
        sx = plsc.load_gather(pos_v, [sv * 2])
        sy = plsc.load_gather(pos_v, [sv * 2 + 1])
        ex = plsc.load_gather(pos_v, [dv * 2])
        ey = plsc.load_gather(pos_v, [dv * 2 + 1])
        dx = ex - sx
        dy = ey - sy
        len2 = dx * dx + dy * dy
        a = jnp.maximum(len2, 1e-12)
        ra = 1.0 / a
        mx = (sx + ex) * 0.5
        my = (sy + ey) * 0.5
        h = 0.5 * _sqrt16(len2)
        sf = sv.astype(jnp.float32)
        df = dv.astype(jnp.float32)
        z = jnp.zeros((16,), jnp.float32)
        fields = [sx, sy, dx, dy, a, ra, mx, my, h, sf, df, z, z, z, z, z]
        eidx = c * 16 + lax.broadcasted_iota(jnp.int32, (16,), 0)
        for fi, val in enumerate(fields):
            fidx = jnp.full((16,), fi, jnp.int32)
            plsc.store_scatter(rowbuf, [eidx, fidx], val)
    pltpu.sync_copy(rowbuf, rows_hbm.at[pl.ds(base, _EPW)])


def _body(rows_ref, out_ref, cols_s, accs_ref, accc_ref):
    i = pl.program_id(0)
    j = pl.program_id(1)

    @pl.when((i == 0) & (j == 0))
    def _init():
        for t in range(E // PT):
            cols_s[:, t * PT:(t + 1) * PT] = rows_ref[t * PT:(t + 1) * PT, :].T
        accs_ref[...] = jnp.zeros((SUB, TCOL), jnp.float32)
        accc_ref[...] = jnp.zeros((SUB, TCOL), jnp.float32)

    @pl.when(j >= i)
    def _compute():
        joff = pl.multiple_of(j * TCOL, TCOL)
        cb = cols_s[:, pl.ds(joff, TCOL)]
        sxj = cb[0:1, :]
        syj = cb[1:2, :]
        dxj = cb[2:3, :]
        dyj = cb[3:4, :]
        ej = cb[4:5, :]
        rej = cb[5:6, :]
        mxj = cb[6:7, :]
        myj = cb[7:8, :]
        hj = cb[8:9, :]
        sfj = cb[9:10, :]
        dfj = cb[10:11, :]
        col_ids = j * TCOL + lax.broadcasted_iota(jnp.int32, (1, TCOL), 1)

        tp = jnp.zeros((SUB, TCOL), jnp.float32)
        tc = jnp.zeros((SUB, TCOL), jnp.float32)
        for k in range(TR // SUB):
            rb = rows_ref[pl.ds(i * TR + k * SUB, SUB), :]
            sxi = rb[:, 0:1]
            syi = rb[:, 1:2]
            dxi = rb[:, 2:3]
            dyi = rb[:, 3:4]
            ai = rb[:, 4:5]
            rai = rb[:, 5:6]
            mxi = rb[:, 6:7]
            myi = rb[:, 7:8]
            hi = rb[:, 8:9]
            sfi = rb[:, 9:10]
            dfi = rb[:, 10:11]

            adj = ((sfi == sfj) | (sfi == dfj) | (dfi == sfj) | (dfi == dfj))
            row_ids = (i * TR + k * SUB
                       + lax.broadcasted_iota(jnp.int32, (SUB, 1), 0))
            tri = col_ids > row_ids
            mdx = mxi - mxj
            mdy = myi - myj
            md2 = mdx * mdx + mdy * mdy
            prox = hi + hj + PROX
            mask = (~adj) & tri & (md2 < prox * prox)
            maskf = mask.astype(jnp.float32)

            b = dxi * dxj + dyi * dyj
            rx = sxi - sxj
            ry = syi - syj
            c = dxi * rx + dyi * ry
            f = dxj * rx + dyj * ry
            rdenom = 1.0 / jnp.maximum(ai * ej - b * b, 1e-12)
            s = jnp.clip((b * f - c * ej) * rdenom, 0.0, 1.0)
            t = jnp.clip((b * s + f) * rej, 0.0, 1.0)
            s = jnp.clip((b * t - c) * rai, 0.0, 1.0)
            ddx = rx + s * dxi - t * dxj
            ddy = ry + s * dyi - t * dyj
            sq = ddx * ddx + ddy * ddy
            dist = jnp.sqrt(jnp.maximum(sq, 1e-24))
            per = jnp.maximum(EPS - dist, 0.0) * maskf

            tp = tp + per
            tc = tc + maskf
        accs_ref[...] += tp
        accc_ref[...] += tc

    @pl.when((i == pl.num_programs(0) - 1) & (j == pl.num_programs(1) - 1))
    def _final():
        total = jnp.sum(accs_ref[...])
        cnt = jnp.sum(accc_ref[...])
        loss = jnp.where(cnt > 0.0, total / jnp.maximum(cnt, 1.0), 0.0)
        out_ref[...] = loss.reshape(1, 1)


def kernel(node_positions, adjacency, edge_index, weight):
    del adjacency, weight
    pos_flat = node_positions.reshape(-1)
    edge_flat = edge_index.reshape(-1)

    sc_prep = pl.kernel(
        _sc_prep,
        out_type=jax.ShapeDtypeStruct((E, NF), jnp.float32),
        mesh=plsc.VectorSubcoreMesh(
            core_axis_name="c", subcore_axis_name="s",
            num_cores=_NC, num_subcores=_NS),
        scratch_types=[
            pltpu.VMEM((2 * E,), jnp.float32),
            pltpu.VMEM((_EPW,), jnp.int32),
            pltpu.VMEM((_EPW,), jnp.int32),
            pltpu.VMEM((_EPW, NF), jnp.float32),
        ],
        compiler_params=pltpu.CompilerParams(
            needs_layout_passes=False, use_tc_tiling_on_sc=False),
    )
    rows = sc_prep(pos_flat, edge_flat)

    loss = pl.pallas_call(
        _body,
        grid=(E // TR, E // TCOL),
        in_specs=[pl.BlockSpec((E, NF), lambda i, j: (0, 0))],
        out_specs=pl.BlockSpec((1, 1), lambda i, j: (0, 0)),
        out_shape=jax.ShapeDtypeStruct((1, 1), jnp.float32),
        scratch_shapes=[
            pltpu.VMEM((NF, E), jnp.float32),
            pltpu.VMEM((SUB, TCOL), jnp.float32),
            pltpu.VMEM((SUB, TCOL), jnp.float32),
        ],
    )(rows)
    return loss.reshape(())
